# baseline (speedup 1.0000x reference)
import functools

import jax
import jax.numpy as jnp
from jax import lax
from jax.experimental import pallas as pl
from jax.experimental.pallas import tpu as pltpu
from jax.experimental.pallas import tpu_sc as plsc

D = 64
SCALE = 8.0
NC, NS, L = 2, 16, 16
NW = NC * NS
B = 4096 * 200
B_PER_W = B // NW
CHUNK = 128
N_CHUNK = B_PER_W // CHUNK


def _emb_body(x_hbm, table_hbm, out_hbm, idx_v, buf, sem):
    wid = lax.axis_index("s") * NC + lax.axis_index("c")
    base = wid * B_PER_W

    pltpu.sync_copy(x_hbm.at[wid], idx_v)

    def chunk_step(j):
        pltpu.async_copy(table_hbm.at[idx_v.at[j]], buf, sem).wait()

        def row_step(r):
            for c in range(D // L):
                sl = pl.ds(c * L, L)
                buf[r, sl] = buf[r, sl] * SCALE

        pl.loop(0, CHUNK)(row_step)

        pltpu.sync_copy(buf, out_hbm.at[pl.ds(base + j * CHUNK, CHUNK)])

    pl.loop(0, N_CHUNK)(chunk_step)


@functools.partial(jax.jit, donate_argnums=())
def _emb_call(x3, table):
    return pl.kernel(
        _emb_body,
        out_type=jax.ShapeDtypeStruct((B, D), jnp.float32),
        mesh=plsc.VectorSubcoreMesh(core_axis_name="c", subcore_axis_name="s"),
        scratch_types=[
            pltpu.VMEM((N_CHUNK, CHUNK), jnp.int32),
            pltpu.VMEM((CHUNK, D), jnp.float32),
            pltpu.SemaphoreType.DMA,
        ],
        compiler_params=pltpu.CompilerParams(use_tc_tiling_on_sc=False),
    )(x3, table)


def kernel(x, table):
    xf = x.reshape(NW, N_CHUNK, CHUNK)
    out = _emb_call(xf, table)
    return out.reshape(x.shape[0], x.shape[1], D)


# speedup vs baseline: 1.2087x; 1.2087x over previous
import functools

import jax
import jax.numpy as jnp
from jax import lax
from jax.experimental import pallas as pl
from jax.experimental.pallas import tpu as pltpu
from jax.experimental.pallas import tpu_sc as plsc

S, T = 4096, 200
D = 64
SCALE = 8.0
NC, NS, L = 2, 16, 16
NW = NC * NS
ROWS_PER_W = S // NW
G0 = 128
G1 = T - G0
N_BUF = 4
LOOK = 2


def _emb_body(x_hbm, table_hbm, out_hbm, idx_v, bufs, gsems, wsems):
    wid = lax.axis_index("s") * NC + lax.axis_index("c")
    base = wid * ROWS_PER_W

    pltpu.sync_copy(x_hbm.at[pl.ds(base, ROWS_PER_W)], idx_v)

    def start_gather(r, b):
        pltpu.async_copy(
            table_hbm.at[idx_v.at[r, pl.ds(0, G0)]],
            bufs.at[b, pl.ds(0, G0)], gsems.at[b])
        pltpu.async_copy(
            table_hbm.at[idx_v.at[r, pl.ds(G0, G1)]],
            bufs.at[b, pl.ds(G0, G1)], gsems.at[b])

    def wait_gather(b):
        pltpu.make_async_copy(
            table_hbm.at[idx_v.at[0, pl.ds(0, G0)]],
            bufs.at[0, pl.ds(0, G0)], gsems.at[b]).wait()
        pltpu.make_async_copy(
            table_hbm.at[idx_v.at[0, pl.ds(G0, G1)]],
            bufs.at[0, pl.ds(G0, G1)], gsems.at[b]).wait()

    def start_write(r, b):
        pltpu.async_copy(bufs.at[b], out_hbm.at[base + r], wsems.at[b])

    def wait_write(b):
        pltpu.make_async_copy(bufs.at[0], out_hbm.at[0], wsems.at[b]).wait()

    for r in range(LOOK):
        start_gather(r, r % N_BUF)

    @pl.loop(0, ROWS_PER_W, step=N_BUF)
    def outer(j0):
        for b in range(N_BUF):
            r = j0 + b
            bl = (b + LOOK) % N_BUF

            @pl.when(r + LOOK < ROWS_PER_W)
            def _():
                @pl.when(r + LOOK >= N_BUF)
                def _():
                    wait_write(bl)
                start_gather(r + LOOK, bl)

            wait_gather(b)

            buf = bufs.at[b]

            @pl.loop(0, T, unroll=8)
            def row_step(rr):
                for c in range(D // L):
                    sl = pl.ds(c * L, L)
                    buf[rr, sl] = buf[rr, sl] * SCALE

            start_write(r, b)

    for b in range(N_BUF):
        wait_write(b)


@jax.jit
def _emb_call(x, table):
    return pl.kernel(
        _emb_body,
        out_type=jax.ShapeDtypeStruct((S, T, D), jnp.float32),
        mesh=plsc.VectorSubcoreMesh(core_axis_name="c", subcore_axis_name="s"),
        scratch_types=[
            pltpu.VMEM((ROWS_PER_W, T), jnp.int32),
            pltpu.VMEM((N_BUF, T, D), jnp.float32),
            pltpu.SemaphoreType.DMA((N_BUF,)),
            pltpu.SemaphoreType.DMA((N_BUF,)),
        ],
        compiler_params=pltpu.CompilerParams(use_tc_tiling_on_sc=False),
    )(x, table)


def kernel(x, table):
    return _emb_call(x, table)
